# baseline (speedup 1.0000x reference)
import functools

import jax
import jax.numpy as jnp
from jax import lax
from jax.experimental import pallas as pl
from jax.experimental.pallas import tpu as pltpu
from jax.experimental.pallas import tpu_sc as plsc

_L = 16


def _make_sc_kernel(B, V, D, NC, NS):
    NW = NC * NS
    b_per_w = B // NW

    mesh = plsc.VectorSubcoreMesh(core_axis_name="c", subcore_axis_name="s")

    @functools.partial(
        pl.kernel,
        mesh=mesh,
        out_type=jax.ShapeDtypeStruct((B,), jnp.float32),
        scratch_types=[
            pltpu.VMEM((b_per_w,), jnp.int32),
            pltpu.VMEM((b_per_w,), jnp.int32),
            pltpu.VMEM((b_per_w,), jnp.float32),
            pltpu.VMEM((b_per_w,), jnp.float32),
            pltpu.VMEM((b_per_w,), jnp.float32),
            pltpu.VMEM_SHARED((V,), jnp.float32),
            pltpu.SemaphoreType.DMA,
            pltpu.SemaphoreType.DMA,
            pltpu.SemaphoreType.DMA,
        ],
    )
    def sc_k(ti_hbm, ci_hbm, tblT_hbm, out_hbm,
             ti_v, ci_v, t_val, c_val, acc_v, col_sh, sem_col, sem_t, sem_c):
        sid = lax.axis_index("s")
        wid = sid * NC + lax.axis_index("c")
        base = wid * b_per_w
        pltpu.sync_copy(ti_hbm.at[pl.ds(base, b_per_w)], ti_v)
        pltpu.sync_copy(ci_hbm.at[pl.ds(base, b_per_w)], ci_v)

        def zero_body(g, carry):
            acc_v[pl.ds(g * _L, _L)] = jnp.zeros((_L,), jnp.float32)
            return carry

        lax.fori_loop(0, b_per_w // _L, zero_body, 0)

        def feat_body(d, carry):
            @pl.when(sid == 0)
            def _stage():
                pltpu.make_async_copy(
                    tblT_hbm.at[d, :], col_sh, sem_col).start()
                pltpu.make_async_copy(
                    tblT_hbm.at[d, :], col_sh, sem_col).wait()

            plsc.subcore_barrier()

            cp_t = pltpu.make_async_copy(col_sh.at[ti_v], t_val, sem_t)
            cp_c = pltpu.make_async_copy(col_sh.at[ci_v], c_val, sem_c)
            cp_t.start()
            cp_c.start()
            cp_t.wait()
            cp_c.wait()

            def mac_body(g, carry2):
                p = pl.ds(g * _L, _L)
                acc_v[p] = acc_v[p] + t_val[p] * c_val[p]
                return carry2

            lax.fori_loop(0, b_per_w // _L, mac_body, 0)
            plsc.subcore_barrier()
            return carry

        lax.fori_loop(0, D, feat_body, 0)

        def sig_body(g, carry):
            p = pl.ds(g * _L, _L)
            acc_v[p] = 1.0 / (1.0 + jnp.exp(-acc_v[p]))
            return carry

        lax.fori_loop(0, b_per_w // _L, sig_body, 0)
        pltpu.sync_copy(acc_v, out_hbm.at[pl.ds(base, b_per_w)])

    return sc_k


def kernel(pair_items, table):
    B = pair_items.shape[1]
    V, D = table.shape
    info = plsc.get_sparse_core_info()
    sc_k = _make_sc_kernel(B, V, D, info.num_cores, info.num_subcores)
    return sc_k(pair_items[0], pair_items[1], table.T)


# speedup vs baseline: 1.7487x; 1.7487x over previous
import functools

import jax
import jax.numpy as jnp
from jax import lax
from jax.experimental import pallas as pl
from jax.experimental.pallas import tpu as pltpu
from jax.experimental.pallas import tpu_sc as plsc

_L = 16


def _make_partial_kernel(B, V, D, NC, NS):
    b_per_s = B // NS
    d_per_c = D // NC

    mesh = plsc.VectorSubcoreMesh(core_axis_name="c", subcore_axis_name="s")

    @functools.partial(
        pl.kernel,
        mesh=mesh,
        out_type=jax.ShapeDtypeStruct((NC * B,), jnp.float32),
        scratch_types=[
            pltpu.VMEM((b_per_s,), jnp.int32),
            pltpu.VMEM((b_per_s,), jnp.int32),
            pltpu.VMEM((b_per_s,), jnp.float32),
            pltpu.VMEM((b_per_s,), jnp.float32),
            pltpu.VMEM((b_per_s,), jnp.float32),
            pltpu.VMEM_SHARED((V,), jnp.float32),
            pltpu.SemaphoreType.DMA,
            pltpu.SemaphoreType.DMA,
        ],
    )
    def sc_k(ti_hbm, ci_hbm, tblT_hbm, out_hbm,
             ti_v, ci_v, t_val, c_val, acc_v, col_sh, sem_t, sem_c):
        sid = lax.axis_index("s")
        cid = lax.axis_index("c")
        base = sid * b_per_s
        pltpu.sync_copy(ti_hbm.at[pl.ds(base, b_per_s)], ti_v)
        pltpu.sync_copy(ci_hbm.at[pl.ds(base, b_per_s)], ci_v)

        def zero_body(g, carry):
            acc_v[pl.ds(g * _L, _L)] = jnp.zeros((_L,), jnp.float32)
            return carry

        lax.fori_loop(0, b_per_s // _L, zero_body, 0)

        def feat_body(dd, carry):
            d = cid * d_per_c + dd

            @pl.when(sid == 0)
            def _stage():
                pltpu.sync_copy(tblT_hbm.at[d, :], col_sh)

            plsc.subcore_barrier()

            cp_t = pltpu.make_async_copy(col_sh.at[ti_v], t_val, sem_t)
            cp_c = pltpu.make_async_copy(col_sh.at[ci_v], c_val, sem_c)
            cp_t.start()
            cp_c.start()
            cp_t.wait()
            cp_c.wait()

            def mac_body(g, carry2):
                p = pl.ds(g * _L, _L)
                acc_v[p] = acc_v[p] + t_val[p] * c_val[p]
                return carry2

            lax.fori_loop(0, b_per_s // _L, mac_body, 0)
            plsc.subcore_barrier()
            return carry

        lax.fori_loop(0, d_per_c, feat_body, 0)
        pltpu.sync_copy(acc_v, out_hbm.at[pl.ds(cid * B + base, b_per_s)])

    return sc_k


def _make_combine_kernel(B, NC, NS):
    NW = NC * NS
    b_per_w = B // NW

    mesh = plsc.VectorSubcoreMesh(core_axis_name="c", subcore_axis_name="s")

    @functools.partial(
        pl.kernel,
        mesh=mesh,
        out_type=jax.ShapeDtypeStruct((B,), jnp.float32),
        scratch_types=[
            pltpu.VMEM((b_per_w,), jnp.float32),
            pltpu.VMEM((b_per_w,), jnp.float32),
        ],
    )
    def cmb_k(part_hbm, out_hbm, p0_v, p1_v):
        wid = lax.axis_index("s") * NC + lax.axis_index("c")
        base = wid * b_per_w
        pltpu.sync_copy(part_hbm.at[pl.ds(base, b_per_w)], p0_v)
        pltpu.sync_copy(part_hbm.at[pl.ds(B + base, b_per_w)], p1_v)

        def body(g, carry):
            p = pl.ds(g * _L, _L)
            s = p0_v[p] + p1_v[p]
            p0_v[p] = 1.0 / (1.0 + jnp.exp(-s))
            return carry

        lax.fori_loop(0, b_per_w // _L, body, 0)
        pltpu.sync_copy(p0_v, out_hbm.at[pl.ds(base, b_per_w)])

    return cmb_k


def kernel(pair_items, table):
    B = pair_items.shape[1]
    V, D = table.shape
    info = plsc.get_sparse_core_info()
    NC, NS = info.num_cores, info.num_subcores
    part_k = _make_partial_kernel(B, V, D, NC, NS)
    cmb_k = _make_combine_kernel(B, NC, NS)
    partials = part_k(pair_items[0], pair_items[1], table.T)
    return cmb_k(partials)


# speedup vs baseline: 1.9924x; 1.1394x over previous
import functools

import jax
import jax.numpy as jnp
from jax import lax
from jax.experimental import pallas as pl
from jax.experimental.pallas import tpu as pltpu
from jax.experimental.pallas import tpu_sc as plsc

_L = 16


def _make_partial_kernel(B, V, D, NC, NS):
    b_per_s = B // NS
    d_per_c = D // NC

    mesh = plsc.VectorSubcoreMesh(core_axis_name="c", subcore_axis_name="s")

    @functools.partial(
        pl.kernel,
        mesh=mesh,
        out_type=jax.ShapeDtypeStruct((NC * B,), jnp.float32),
        scratch_types=[
            pltpu.VMEM((b_per_s,), jnp.int32),
            pltpu.VMEM((b_per_s,), jnp.int32),
            pltpu.VMEM((b_per_s,), jnp.float32),
            pltpu.VMEM((b_per_s,), jnp.float32),
            pltpu.VMEM((b_per_s,), jnp.float32),
            pltpu.VMEM_SHARED((V,), jnp.float32),
            pltpu.VMEM_SHARED((V,), jnp.float32),
            pltpu.SemaphoreType.DMA,
            pltpu.SemaphoreType.DMA,
            pltpu.SemaphoreType.DMA,
            pltpu.SemaphoreType.DMA,
        ],
    )
    def sc_k(ti_hbm, ci_hbm, tblT_hbm, out_hbm,
             ti_v, ci_v, t_val, c_val, acc_v, col0_sh, col1_sh,
             sem_s0, sem_s1, sem_t, sem_c):
        sid = lax.axis_index("s")
        cid = lax.axis_index("c")
        base = sid * b_per_s
        pltpu.sync_copy(ti_hbm.at[pl.ds(base, b_per_s)], ti_v)
        pltpu.sync_copy(ci_hbm.at[pl.ds(base, b_per_s)], ci_v)

        def zero_body(g, carry):
            acc_v[pl.ds(g * _L, _L)] = jnp.zeros((_L,), jnp.float32)
            return carry

        lax.fori_loop(0, b_per_s // _L, zero_body, 0)

        d0 = cid * d_per_c

        @pl.when(sid == 0)
        def _prologue():
            pltpu.make_async_copy(tblT_hbm.at[d0, :], col0_sh, sem_s0).start()

        def feat_pair(dd2, carry):
            for par, colP, colO, semP, semO in (
                (0, col0_sh, col1_sh, sem_s0, sem_s1),
                (1, col1_sh, col0_sh, sem_s1, sem_s0),
            ):
                dd = 2 * dd2 + par
                d = d0 + dd

                @pl.when(sid == 0)
                def _wait_stage():
                    pltpu.make_async_copy(
                        tblT_hbm.at[d, :], colP, semP).wait()

                plsc.subcore_barrier()

                @pl.when(jnp.logical_and(sid == 0, dd + 1 < d_per_c))
                def _stage_next():
                    pltpu.make_async_copy(
                        tblT_hbm.at[d + 1, :], colO, semO).start()

                cp_t = pltpu.make_async_copy(colP.at[ti_v], t_val, sem_t)
                cp_c = pltpu.make_async_copy(colP.at[ci_v], c_val, sem_c)
                cp_t.start()
                cp_c.start()
                cp_t.wait()
                cp_c.wait()

                def mac_body(g, carry2):
                    p = pl.ds(g * _L, _L)
                    acc_v[p] = acc_v[p] + t_val[p] * c_val[p]
                    return carry2

                lax.fori_loop(0, b_per_s // _L, mac_body, 0)
                plsc.subcore_barrier()
            return carry

        lax.fori_loop(0, d_per_c // 2, feat_pair, 0)
        pltpu.sync_copy(acc_v, out_hbm.at[pl.ds(cid * B + base, b_per_s)])

    return sc_k


def _make_combine_kernel(B, NC, NS):
    NW = NC * NS
    b_per_w = B // NW

    mesh = plsc.VectorSubcoreMesh(core_axis_name="c", subcore_axis_name="s")

    @functools.partial(
        pl.kernel,
        mesh=mesh,
        out_type=jax.ShapeDtypeStruct((B,), jnp.float32),
        scratch_types=[
            pltpu.VMEM((b_per_w,), jnp.float32),
            pltpu.VMEM((b_per_w,), jnp.float32),
        ],
    )
    def cmb_k(part_hbm, out_hbm, p0_v, p1_v):
        wid = lax.axis_index("s") * NC + lax.axis_index("c")
        base = wid * b_per_w
        pltpu.sync_copy(part_hbm.at[pl.ds(base, b_per_w)], p0_v)
        pltpu.sync_copy(part_hbm.at[pl.ds(B + base, b_per_w)], p1_v)

        def body(g, carry):
            p = pl.ds(g * _L, _L)
            s = p0_v[p] + p1_v[p]
            p0_v[p] = 1.0 / (1.0 + jnp.exp(-s))
            return carry

        lax.fori_loop(0, b_per_w // _L, body, 0)
        pltpu.sync_copy(p0_v, out_hbm.at[pl.ds(base, b_per_w)])

    return cmb_k


def kernel(pair_items, table):
    B = pair_items.shape[1]
    V, D = table.shape
    info = plsc.get_sparse_core_info()
    NC, NS = info.num_cores, info.num_subcores
    part_k = _make_partial_kernel(B, V, D, NC, NS)
    cmb_k = _make_combine_kernel(B, NC, NS)
    partials = part_k(pair_items[0], pair_items[1], table.T)
    return cmb_k(partials)
